# baseline (speedup 1.0000x reference)
import jax
import jax.numpy as jnp
from jax import lax
from jax.experimental import pallas as pl

N = 20000
C = 81
B = 1000


def _body(lg_ref, bb_ref, lb_ref, tg_ref, ce_ref, bx_ref):
    i = pl.program_id(0)
    lg = lg_ref[...]
    lb = lb_ref[...]
    m = jnp.max(lg, axis=1, keepdims=True)
    e = jnp.exp(lg - m)
    s = jnp.sum(e, axis=1, keepdims=True)
    lse = jnp.log(s) + m
    cls_iota = lax.broadcasted_iota(jnp.int32, (B, C), 1)
    lab_logit = jnp.sum(jnp.where(cls_iota == lb, lg, 0.0), axis=1,
                        keepdims=True)
    ce_part = jnp.sum(lse - lab_logit).reshape(1, 1)

    bb = bb_ref[...]
    col_iota = lax.broadcasted_iota(jnp.int32, (B, 4 * C), 1)
    pos = lb > 0
    cmask = ((col_iota >> 2) == lb) & pos
    mb = jnp.where(cmask, bb, 0.0)
    sel = (lax.broadcasted_iota(jnp.int32, (4 * C, 4), 0) % 4
           ) == lax.broadcasted_iota(jnp.int32, (4 * C, 4), 1)
    pred = jnp.dot(mb, sel.astype(jnp.float32),
                   preferred_element_type=jnp.float32)
    d = jnp.abs(pred - tg_ref[...])
    pe = jnp.where(d < 1.0, 0.5 * d * d, d - 0.5)
    bx_part = jnp.sum(jnp.where(pos, pe, 0.0)).reshape(1, 1)

    @pl.when(i == 0)
    def _init():
        ce_ref[...] = jnp.zeros((1, 1), jnp.float32)
        bx_ref[...] = jnp.zeros((1, 1), jnp.float32)

    ce_ref[...] += ce_part
    bx_ref[...] += bx_part


def kernel(logits, bbox_reg, labels, regression_targets):
    grid = N // B
    ce, bx = pl.pallas_call(
        _body,
        grid=(grid,),
        in_specs=[
            pl.BlockSpec((B, C), lambda i: (i, 0)),
            pl.BlockSpec((B, 4 * C), lambda i: (i, 0)),
            pl.BlockSpec((B, 1), lambda i: (i, 0)),
            pl.BlockSpec((B, 4), lambda i: (i, 0)),
        ],
        out_specs=[
            pl.BlockSpec((1, 1), lambda i: (0, 0)),
            pl.BlockSpec((1, 1), lambda i: (0, 0)),
        ],
        out_shape=[
            jax.ShapeDtypeStruct((1, 1), jnp.float32),
            jax.ShapeDtypeStruct((1, 1), jnp.float32),
        ],
    )(logits, bbox_reg, labels.reshape(N, 1), regression_targets)
    return (ce[0, 0] / N, bx[0, 0] / N)
